# baseline (speedup 1.0000x reference)
import functools

import jax
import jax.numpy as jnp
from jax import lax
from jax.experimental import pallas as pl
from jax.experimental.pallas import tpu as pltpu
from jax.experimental.pallas import tpu_sc as plsc

N = 50000
E = 800000
RD = 16
NSC = 32
NCLS = 6
NL = 3
NG = 512

NCORES = 2
NTILES = 16
NW = NCORES * NTILES
CH = 128
CPT = 200
E_PAD = NW * CPT * CH
N_PAD = 50048
NPT = N_PAD // NTILES
ZR = 68
IBG = 8
NSLABG = CPT // IBG
IBL = 4
NSLABL = CPT // IBL

BLK = 3128
NBLK = N_PAD // BLK
EBLK = 4096
NEBLK = E_PAD // EBLK

_f32 = jnp.float32
PW = 16


def _graph_sums(posw, batchp):
    def body(b_ref, p_ref, s_ref):
        i = pl.program_id(0)
        oh = (b_ref[...] == lax.broadcasted_iota(jnp.int32, (BLK, NG), 1))
        part = lax.dot_general(oh.astype(_f32), p_ref[...],
                               (((0,), (0,)), ((), ())),
                               preferred_element_type=_f32)

        @pl.when(i == 0)
        def _():
            s_ref[...] = part

        @pl.when(i > 0)
        def _():
            s_ref[...] += part

    return pl.pallas_call(
        body,
        grid=(NBLK,),
        in_specs=[pl.BlockSpec((BLK, 1), lambda i: (i, 0)),
                  pl.BlockSpec((BLK, PW), lambda i: (i, 0))],
        out_specs=pl.BlockSpec((NG, PW), lambda i: (0, 0)),
        out_shape=jax.ShapeDtypeStruct((NG, PW), _f32),
    )(batchp, posw)


def _center(sums, posw, batchp):
    def body(s_ref, b_ref, p_ref, o_ref):
        s = s_ref[...]
        cnt = jnp.maximum(s[:, 3:4], 1.0)
        cmask = (lax.broadcasted_iota(jnp.int32, (NG, PW), 1) < 3).astype(_f32)
        meanw = (s / cnt) * cmask
        oh = (b_ref[...] == lax.broadcasted_iota(jnp.int32, (BLK, NG), 1))
        o_ref[...] = p_ref[...] - jnp.dot(oh.astype(_f32), meanw,
                                          preferred_element_type=_f32)

    return pl.pallas_call(
        body,
        grid=(NBLK,),
        in_specs=[pl.BlockSpec((NG, PW), lambda i: (0, 0)),
                  pl.BlockSpec((BLK, 1), lambda i: (i, 0)),
                  pl.BlockSpec((BLK, PW), lambda i: (i, 0))],
        out_specs=pl.BlockSpec((BLK, PW), lambda i: (i, 0)),
        out_shape=jax.ShapeDtypeStruct((N_PAD, PW), _f32),
    )(sums, batchp, posw)


def _edge_he(dv, emask, w_emb, b_emb):
    def body(dv_ref, em_ref, we_ref, be_ref, he_ref):
        d = dv_ref[...]
        cmask = (lax.broadcasted_iota(jnp.int32, (EBLK, PW), 1) < 3).astype(_f32)
        dist = jnp.sqrt(jnp.sum(d * d * cmask, axis=1, keepdims=True))
        he_ref[...] = (jnp.maximum(dist * we_ref[...] + be_ref[...], 0.0)
                       * em_ref[...])

    bspec = pl.BlockSpec((1, NSC), lambda i: (0, 0))
    return pl.pallas_call(
        body,
        grid=(NEBLK,),
        in_specs=[pl.BlockSpec((EBLK, PW), lambda i: (i, 0)),
                  pl.BlockSpec((EBLK, 1), lambda i: (i, 0)),
                  bspec, bspec],
        out_specs=pl.BlockSpec((EBLK, NSC), lambda i: (i, 0)),
        out_shape=jax.ShapeDtypeStruct((E_PAD, NSC), _f32),
    )(dv, emask, w_emb, b_emb)


def _edge_c(dv, emask, wr, br):
    def body(dv_ref, em_ref, w1_ref, w2_ref, w3_ref, b1_ref, b2_ref, b3_ref,
             c1_ref, c2_ref, c3_ref):
        d = dv_ref[...]
        cmask = (lax.broadcasted_iota(jnp.int32, (EBLK, PW), 1) < 3).astype(_f32)
        dist = jnp.sqrt(jnp.sum(d * d * cmask, axis=1, keepdims=True))
        em = em_ref[...]
        dd = dist + 1e-6
        nvec = (lax.broadcasted_iota(jnp.int32, (EBLK, RD), 1) + 1).astype(_f32)
        rbf = jnp.sqrt(2.0) * jnp.sin(nvec * jnp.pi * dd) / dd
        c1_ref[...] = (jnp.dot(rbf, w1_ref[...], preferred_element_type=_f32)
                       + b1_ref[...]) * em
        c2_ref[...] = (jnp.dot(rbf, w2_ref[...], preferred_element_type=_f32)
                       + b2_ref[...]) * em
        c3_ref[...] = (jnp.dot(rbf, w3_ref[...], preferred_element_type=_f32)
                       + b3_ref[...]) * em

    wspec = pl.BlockSpec((RD, NSC), lambda i: (0, 0))
    bspec = pl.BlockSpec((1, NSC), lambda i: (0, 0))
    eo = pl.BlockSpec((EBLK, NSC), lambda i: (i, 0))
    return pl.pallas_call(
        body,
        grid=(NEBLK,),
        in_specs=[pl.BlockSpec((EBLK, PW), lambda i: (i, 0)),
                  pl.BlockSpec((EBLK, 1), lambda i: (i, 0)),
                  wspec, wspec, wspec, bspec, bspec, bspec],
        out_specs=[eo, eo, eo],
        out_shape=[jax.ShapeDtypeStruct((E_PAD, NSC), _f32)] * 3,
    )(dv, emask, wr[0], wr[1], wr[2], br[0], br[1], br[2])


def _combine0_proj(p0, p1, wd, ws):
    def body(p0_ref, p1_ref, wd_ref, ws_ref, x_ref, a_ref, b_ref):
        x = p0_ref[...] + p1_ref[...]
        x_ref[...] = x
        a_ref[...] = jnp.dot(x, wd_ref[...], preferred_element_type=_f32)
        b_ref[...] = jnp.dot(x, ws_ref[...], preferred_element_type=_f32)

    nb = pl.BlockSpec((BLK, NSC), lambda i: (i, 0))
    ws_ = pl.BlockSpec((NSC, NSC), lambda i: (0, 0))
    return pl.pallas_call(
        body, grid=(NBLK,), in_specs=[nb, nb, ws_, ws_],
        out_specs=[nb, nb, nb],
        out_shape=[jax.ShapeDtypeStruct((N_PAD, NSC), _f32)] * 3,
    )(p0, p1, wd, ws)


def _update_proj(xprev, p0, p1, wu, bu, wd, ws):
    def body(x_ref, p0_ref, p1_ref, wu_ref, bu_ref, wd_ref, ws_ref,
             o_ref, a_ref, b_ref):
        agg = p0_ref[...] + p1_ref[...]
        x = x_ref[...] + jnp.maximum(
            jnp.dot(agg, wu_ref[...], preferred_element_type=_f32)
            + bu_ref[...], 0.0)
        o_ref[...] = x
        a_ref[...] = jnp.dot(x, wd_ref[...], preferred_element_type=_f32)
        b_ref[...] = jnp.dot(x, ws_ref[...], preferred_element_type=_f32)

    nb = pl.BlockSpec((BLK, NSC), lambda i: (i, 0))
    ws_ = pl.BlockSpec((NSC, NSC), lambda i: (0, 0))
    return pl.pallas_call(
        body,
        grid=(NBLK,),
        in_specs=[nb, nb, nb, ws_, pl.BlockSpec((1, NSC), lambda i: (0, 0)),
                  ws_, ws_],
        out_specs=[nb, nb, nb],
        out_shape=[jax.ShapeDtypeStruct((N_PAD, NSC), _f32)] * 3,
    )(xprev, p0, p1, wu, bu, wd, ws)


def _readout(xprev, p0, p1, wu, bu, w1, b1, w2, b2, batchp, nmask):
    def body(x_ref, p0_ref, p1_ref, wu_ref, bu_ref, w1_ref, b1_ref,
             w2_ref, b2_ref, b_ref, nm_ref, o_ref):
        i = pl.program_id(0)
        agg = p0_ref[...] + p1_ref[...]
        x = x_ref[...] + jnp.maximum(
            jnp.dot(agg, wu_ref[...], preferred_element_type=_f32)
            + bu_ref[...], 0.0)
        h = jnp.maximum(
            jnp.dot(x, w1_ref[...], preferred_element_type=_f32)
            + b1_ref[...], 0.0)
        u = (jnp.dot(h, w2_ref[...], preferred_element_type=_f32)
             + b2_ref[...]) * nm_ref[...]
        oh = (b_ref[...] == lax.broadcasted_iota(jnp.int32, (BLK, NG), 1))
        part = lax.dot_general(oh.astype(_f32), u,
                               (((0,), (0,)), ((), ())),
                               preferred_element_type=_f32)

        @pl.when(i == 0)
        def _():
            o_ref[...] = part

        @pl.when(i > 0)
        def _():
            o_ref[...] += part

    nb = pl.BlockSpec((BLK, NSC), lambda i: (i, 0))
    return pl.pallas_call(
        body,
        grid=(NBLK,),
        in_specs=[nb, nb, nb,
                  pl.BlockSpec((NSC, NSC), lambda i: (0, 0)),
                  pl.BlockSpec((1, NSC), lambda i: (0, 0)),
                  pl.BlockSpec((NSC, 3 * NSC), lambda i: (0, 0)),
                  pl.BlockSpec((1, 3 * NSC), lambda i: (0, 0)),
                  pl.BlockSpec((3 * NSC, NCLS), lambda i: (0, 0)),
                  pl.BlockSpec((1, NCLS), lambda i: (0, 0)),
                  pl.BlockSpec((BLK, 1), lambda i: (i, 0)),
                  pl.BlockSpec((BLK, 1), lambda i: (i, 0))],
        out_specs=pl.BlockSpec((NG, NCLS), lambda i: (0, 0)),
        out_shape=jax.ShapeDtypeStruct((NG, NCLS), _f32),
    )(xprev, p0, p1, wu, bu, w1, b1, w2, b2, batchp, nmask)


def _sc_mesh():
    return plsc.VectorSubcoreMesh(core_axis_name="c", subcore_axis_name="s")


_SC_PARAMS = pltpu.CompilerParams(use_tc_tiling_on_sc=False)


def _sc_gather_pos(pc, row2d, col2d):
    @functools.partial(
        pl.kernel,
        out_type=jax.ShapeDtypeStruct((E_PAD, PW), _f32),
        mesh=_sc_mesh(),
        compiler_params=_SC_PARAMS,
        scratch_types=[pltpu.VMEM((IBG, CH), jnp.int32),
                       pltpu.VMEM((IBG, CH), jnp.int32),
                       pltpu.VMEM((CH, PW), _f32),
                       pltpu.VMEM((CH, PW), _f32),
                       pltpu.VMEM((CH, PW), _f32),
                       pltpu.VMEM((CH, PW), _f32),
                       pltpu.SemaphoreType.DMA,
                       pltpu.SemaphoreType.DMA,
                       pltpu.SemaphoreType.DMA,
                       pltpu.SemaphoreType.DMA])
    def k(pc_hbm, row_hbm, col_hbm, dv_hbm,
          ridx, cidx, rb0, rb1, cb0, cb1, s10, s11, s20, s21):
        wid = lax.axis_index("c") * NTILES + lax.axis_index("s")
        rbs, cbs = (rb0, rb1), (cb0, cb1)
        s1s, s2s = (s10, s11), (s20, s21)

        @pl.loop(0, NSLABG)
        def _(t):
            pltpu.sync_copy(row_hbm.at[pl.ds(wid * CPT + t * IBG, IBG)], ridx)
            pltpu.sync_copy(col_hbm.at[pl.ds(wid * CPT + t * IBG, IBG)], cidx)

            def start(b):
                sl = b % 2
                da = pltpu.async_copy(pc_hbm.at[ridx.at[b]], rbs[sl], s1s[sl])
                db = pltpu.async_copy(pc_hbm.at[cidx.at[b]], cbs[sl], s2s[sl])
                return (da, db)

            pend = start(0)
            for b in range(IBG):
                sl = b % 2
                nxt = start(b + 1) if b + 1 < IBG else None
                for d in pend:
                    d.wait()
                rb, cb = rbs[sl], cbs[sl]

                @pl.loop(0, CH)
                def _(r):
                    s_ = pl.ds(0, 16)
                    rb[r, s_] = rb[r, s_] - cb[r, s_]

                base = (wid * CPT + t * IBG + b) * CH
                pltpu.sync_copy(rb, dv_hbm.at[pl.ds(base, CH)])
                pend = nxt

    return k(pc, row2d, col2d)


def _zero_acc(zb, acc, sid):
    zeros16 = jnp.zeros((16,), _f32)

    @pl.loop(0, ZR)
    def _(r):
        zb[r, pl.ds(0, 16)] = zeros16
        zb[r, pl.ds(16, 16)] = zeros16

    @pl.loop(0, NPT // ZR)
    def _(t):
        pltpu.sync_copy(zb, acc.at[pl.ds(sid * NPT + t * ZR, ZR)])


def _sc_scatter_he(he, row2d):
    @functools.partial(
        pl.kernel,
        out_type=jax.ShapeDtypeStruct((NCORES, N_PAD, NSC), _f32),
        mesh=_sc_mesh(),
        compiler_params=_SC_PARAMS,
        scratch_types=[pltpu.VMEM((IBG, CH), jnp.int32),
                       pltpu.VMEM((CH, NSC), _f32),
                       pltpu.VMEM((CH, NSC), _f32),
                       pltpu.VMEM((ZR, NSC), _f32),
                       pltpu.VMEM_SHARED((N_PAD, NSC), _f32),
                       pltpu.SemaphoreType.DMA,
                       pltpu.SemaphoreType.DMA])
    def k(he_hbm, row_hbm, out_hbm, ridx, hb0, hb1, zb, acc, s0, s1):
        cid = lax.axis_index("c")
        sid = lax.axis_index("s")
        wid = cid * NTILES + sid
        hbs, ss = (hb0, hb1), (s0, s1)
        _zero_acc(zb, acc, sid)
        plsc.subcore_barrier()

        @pl.loop(0, NSLABG)
        def _(t):
            pltpu.sync_copy(row_hbm.at[pl.ds(wid * CPT + t * IBG, IBG)], ridx)

            def start(b):
                sl = b % 2
                base = (wid * CPT + t * IBG + b) * CH
                return pltpu.async_copy(he_hbm.at[pl.ds(base, CH)],
                                        hbs[sl], ss[sl])

            pend = start(0)
            for b in range(IBG):
                sl = b % 2
                nxt = start(b + 1) if b + 1 < IBG else None
                pend.wait()
                pltpu.sync_copy(hbs[sl], acc.at[ridx.at[b]], add=True)
                pend = nxt

        plsc.subcore_barrier()
        pltpu.sync_copy(acc.at[pl.ds(sid * NPT, NPT)],
                        out_hbm.at[cid].at[pl.ds(sid * NPT, NPT)])

    return k(he, row2d)


def _sc_layer(a, b, c, row2d, col2d):
    @functools.partial(
        pl.kernel,
        out_type=jax.ShapeDtypeStruct((NCORES, N_PAD, NSC), _f32),
        mesh=_sc_mesh(),
        compiler_params=_SC_PARAMS,
        scratch_types=[pltpu.VMEM((IBL, CH), jnp.int32),
                       pltpu.VMEM((IBL, CH), jnp.int32),
                       pltpu.VMEM((CH, NSC), _f32),
                       pltpu.VMEM((CH, NSC), _f32),
                       pltpu.VMEM((CH, NSC), _f32),
                       pltpu.VMEM((CH, NSC), _f32),
                       pltpu.VMEM((CH, NSC), _f32),
                       pltpu.VMEM((CH, NSC), _f32),
                       pltpu.VMEM((ZR, NSC), _f32),
                       pltpu.VMEM_SHARED((N_PAD, NSC), _f32),
                       pltpu.SemaphoreType.DMA,
                       pltpu.SemaphoreType.DMA,
                       pltpu.SemaphoreType.DMA,
                       pltpu.SemaphoreType.DMA,
                       pltpu.SemaphoreType.DMA,
                       pltpu.SemaphoreType.DMA])
    def k(a_hbm, b_hbm, c_hbm, row_hbm, col_hbm, out_hbm,
          ridx, cidx, ab0, ab1, bb0, bb1, cb0, cb1, zb, acc,
          sa0, sa1, sb0, sb1, sc0, sc1):
        cid = lax.axis_index("c")
        sid = lax.axis_index("s")
        wid = cid * NTILES + sid
        abs_, bbs, cbs = (ab0, ab1), (bb0, bb1), (cb0, cb1)
        sas, sbs, scs = (sa0, sa1), (sb0, sb1), (sc0, sc1)
        _zero_acc(zb, acc, sid)
        plsc.subcore_barrier()

        @pl.loop(0, NSLABL)
        def _(t):
            pltpu.sync_copy(row_hbm.at[pl.ds(wid * CPT + t * IBL, IBL)], ridx)
            pltpu.sync_copy(col_hbm.at[pl.ds(wid * CPT + t * IBL, IBL)], cidx)

            def start(b):
                sl = b % 2
                base = (wid * CPT + t * IBL + b) * CH
                da = pltpu.async_copy(a_hbm.at[ridx.at[b]], abs_[sl], sas[sl])
                db = pltpu.async_copy(b_hbm.at[cidx.at[b]], bbs[sl], sbs[sl])
                dc = pltpu.async_copy(c_hbm.at[pl.ds(base, CH)],
                                      cbs[sl], scs[sl])
                return (da, db, dc)

            pend = start(0)
            for b in range(IBL):
                sl = b % 2
                nxt = start(b + 1) if b + 1 < IBL else None
                for d in pend:
                    d.wait()
                ab, bb, cb = abs_[sl], bbs[sl], cbs[sl]

                @pl.loop(0, CH)
                def _(r):
                    for c0 in (0, 16):
                        s_ = pl.ds(c0, 16)
                        ab[r, s_] = jnp.maximum(
                            ab[r, s_] + bb[r, s_] + cb[r, s_], 0.0)

                pltpu.sync_copy(ab, acc.at[ridx.at[b]], add=True)
                pend = nxt

        plsc.subcore_barrier()
        pltpu.sync_copy(acc.at[pl.ds(sid * NPT, NPT)],
                        out_hbm.at[cid].at[pl.ds(sid * NPT, NPT)])

    return k(a, b, c, row2d, col2d)


def kernel(pos, edge_index, batch, W_emb, b_emb, W_msg, b_msg, W_upd, b_upd,
           W1, b1, W2, b2):
    row = edge_index[0].astype(jnp.int32)
    col = edge_index[1].astype(jnp.int32)
    batch = batch.astype(jnp.int32)

    ech = E // CH
    padc = jnp.full((E_PAD // CH - ech, CH), N, jnp.int32)
    row2d = jnp.concatenate([row.reshape(ech, CH), padc])
    col2d = jnp.concatenate([col.reshape(ech, CH), padc])

    posw = jnp.zeros((N_PAD, PW), _f32)
    posw = posw.at[:N, :3].set(pos).at[:N, 3].set(1.0)
    batchp = jnp.zeros((N_PAD, 1), jnp.int32).at[:N, 0].set(batch)
    nmask = (jnp.arange(N_PAD) < N).astype(_f32)[:, None]
    emask = (jnp.arange(E_PAD) < E).astype(_f32)[:, None]

    w_emb = W_emb.reshape(1, NSC)
    b_emb2 = b_emb.reshape(1, NSC)
    wd = [W_msg[i, :NSC, :] for i in range(NL)]
    ws = [W_msg[i, NSC:2 * NSC, :] for i in range(NL)]
    wr = [W_msg[i, 2 * NSC:, :] for i in range(NL)]
    bm = [b_msg[i].reshape(1, NSC) for i in range(NL)]
    wu = [W_upd[i] for i in range(NL)]
    bu = [b_upd[i].reshape(1, NSC) for i in range(NL)]
    b1_2 = b1.reshape(1, 3 * NSC)
    b2_2 = b2.reshape(1, NCLS)

    sums = _graph_sums(posw, batchp)
    pc = _center(sums, posw, batchp)

    dv = _sc_gather_pos(pc, row2d, col2d)
    he = _edge_he(dv, emask, w_emb, b_emb2)

    parts = _sc_scatter_he(he, row2d)
    cs = _edge_c(dv, emask, wr, bm)
    x, a_n, b_n = _combine0_proj(parts[0], parts[1], wd[0], ws[0])

    for i in range(NL):
        parts = _sc_layer(a_n, b_n, cs[i], row2d, col2d)
        if i < NL - 1:
            x, a_n, b_n = _update_proj(x, parts[0], parts[1], wu[i], bu[i],
                                       wd[i + 1], ws[i + 1])
        else:
            out = _readout(x, parts[0], parts[1], wu[i], bu[i],
                           W1, b1_2, W2, b2_2, batchp, nmask)
    return out


# speedup vs baseline: 1.0727x; 1.0727x over previous
import functools

import jax
import jax.numpy as jnp
from jax import lax
from jax.experimental import pallas as pl
from jax.experimental.pallas import tpu as pltpu
from jax.experimental.pallas import tpu_sc as plsc

N = 50000
E = 800000
RD = 16
NSC = 32
NCLS = 6
NL = 3
NG = 512

NCORES = 2
NTILES = 16
NW = NCORES * NTILES
CH = 128
CPT = 200
E_PAD = NW * CPT * CH
N_PAD = 50048
NPT = N_PAD // NTILES
ZR = 68
IBG = 8
NSLABG = CPT // IBG
IBL = 4
NSLABL = CPT // IBL

BLK = 3128
NBLK = N_PAD // BLK
EBLK = 4096
NEBLK = E_PAD // EBLK

_f32 = jnp.float32
PW = 16


def _graph_sums(posw, batchp):
    def body(b_ref, p_ref, s_ref):
        i = pl.program_id(0)
        oh = (b_ref[...] == lax.broadcasted_iota(jnp.int32, (BLK, NG), 1))
        part = lax.dot_general(oh.astype(_f32), p_ref[...],
                               (((0,), (0,)), ((), ())),
                               preferred_element_type=_f32)

        @pl.when(i == 0)
        def _():
            s_ref[...] = part

        @pl.when(i > 0)
        def _():
            s_ref[...] += part

    return pl.pallas_call(
        body,
        grid=(NBLK,),
        in_specs=[pl.BlockSpec((BLK, 1), lambda i: (i, 0)),
                  pl.BlockSpec((BLK, PW), lambda i: (i, 0))],
        out_specs=pl.BlockSpec((NG, PW), lambda i: (0, 0)),
        out_shape=jax.ShapeDtypeStruct((NG, PW), _f32),
    )(batchp, posw)


def _center(sums, posw, batchp):
    def body(s_ref, b_ref, p_ref, o_ref):
        s = s_ref[...]
        cnt = jnp.maximum(s[:, 3:4], 1.0)
        cmask = (lax.broadcasted_iota(jnp.int32, (NG, PW), 1) < 3).astype(_f32)
        meanw = (s / cnt) * cmask
        oh = (b_ref[...] == lax.broadcasted_iota(jnp.int32, (BLK, NG), 1))
        o_ref[...] = p_ref[...] - jnp.dot(oh.astype(_f32), meanw,
                                          preferred_element_type=_f32)

    return pl.pallas_call(
        body,
        grid=(NBLK,),
        in_specs=[pl.BlockSpec((NG, PW), lambda i: (0, 0)),
                  pl.BlockSpec((BLK, 1), lambda i: (i, 0)),
                  pl.BlockSpec((BLK, PW), lambda i: (i, 0))],
        out_specs=pl.BlockSpec((BLK, PW), lambda i: (i, 0)),
        out_shape=jax.ShapeDtypeStruct((N_PAD, PW), _f32),
    )(sums, batchp, posw)


def _edge_feats(dv, emask, w_emb, b_emb, wr, br):
    def body(dv_ref, em_ref, we_ref, be_ref,
             w1_ref, w2_ref, w3_ref, b1_ref, b2_ref, b3_ref,
             he_ref, c1_ref, c2_ref, c3_ref):
        d = dv_ref[...]
        cmask = (lax.broadcasted_iota(jnp.int32, (EBLK, PW), 1) < 3).astype(_f32)
        d2 = jnp.sum(d * d * cmask, axis=1, keepdims=True)
        dist = jnp.sqrt(d2)
        em = em_ref[...]
        he_ref[...] = jnp.maximum(dist * we_ref[...] + be_ref[...], 0.0) * em
        dd = dist + 1e-6
        nvec = (lax.broadcasted_iota(jnp.int32, (EBLK, RD), 1) + 1).astype(_f32)
        rbf = jnp.sqrt(2.0) * jnp.sin(nvec * jnp.pi * dd) / dd
        c1_ref[...] = (jnp.dot(rbf, w1_ref[...], preferred_element_type=_f32)
                       + b1_ref[...]) * em
        c2_ref[...] = (jnp.dot(rbf, w2_ref[...], preferred_element_type=_f32)
                       + b2_ref[...]) * em
        c3_ref[...] = (jnp.dot(rbf, w3_ref[...], preferred_element_type=_f32)
                       + b3_ref[...]) * em

    wspec = pl.BlockSpec((RD, NSC), lambda i: (0, 0))
    bspec = pl.BlockSpec((1, NSC), lambda i: (0, 0))
    eb = pl.BlockSpec((EBLK, PW), lambda i: (i, 0))
    eo = pl.BlockSpec((EBLK, NSC), lambda i: (i, 0))
    return pl.pallas_call(
        body,
        grid=(NEBLK,),
        in_specs=[eb, pl.BlockSpec((EBLK, 1), lambda i: (i, 0)),
                  bspec, bspec, wspec, wspec, wspec, bspec, bspec, bspec],
        out_specs=[eo, eo, eo, eo],
        out_shape=[jax.ShapeDtypeStruct((E_PAD, NSC), _f32)] * 4,
    )(dv, emask, w_emb, b_emb, wr[0], wr[1], wr[2], br[0], br[1], br[2])


def _combine0_proj(p0, p1, wd, ws):
    def body(p0_ref, p1_ref, wd_ref, ws_ref, x_ref, a_ref, b_ref):
        x = p0_ref[...] + p1_ref[...]
        x_ref[...] = x
        a_ref[...] = jnp.dot(x, wd_ref[...], preferred_element_type=_f32)
        b_ref[...] = jnp.dot(x, ws_ref[...], preferred_element_type=_f32)

    nb = pl.BlockSpec((BLK, NSC), lambda i: (i, 0))
    ws_ = pl.BlockSpec((NSC, NSC), lambda i: (0, 0))
    return pl.pallas_call(
        body, grid=(NBLK,), in_specs=[nb, nb, ws_, ws_],
        out_specs=[nb, nb, nb],
        out_shape=[jax.ShapeDtypeStruct((N_PAD, NSC), _f32)] * 3,
    )(p0, p1, wd, ws)


def _update_proj(xprev, p0, p1, wu, bu, wd, ws):
    def body(x_ref, p0_ref, p1_ref, wu_ref, bu_ref, wd_ref, ws_ref,
             o_ref, a_ref, b_ref):
        agg = p0_ref[...] + p1_ref[...]
        x = x_ref[...] + jnp.maximum(
            jnp.dot(agg, wu_ref[...], preferred_element_type=_f32)
            + bu_ref[...], 0.0)
        o_ref[...] = x
        a_ref[...] = jnp.dot(x, wd_ref[...], preferred_element_type=_f32)
        b_ref[...] = jnp.dot(x, ws_ref[...], preferred_element_type=_f32)

    nb = pl.BlockSpec((BLK, NSC), lambda i: (i, 0))
    ws_ = pl.BlockSpec((NSC, NSC), lambda i: (0, 0))
    return pl.pallas_call(
        body,
        grid=(NBLK,),
        in_specs=[nb, nb, nb, ws_, pl.BlockSpec((1, NSC), lambda i: (0, 0)),
                  ws_, ws_],
        out_specs=[nb, nb, nb],
        out_shape=[jax.ShapeDtypeStruct((N_PAD, NSC), _f32)] * 3,
    )(xprev, p0, p1, wu, bu, wd, ws)


def _readout(xprev, p0, p1, wu, bu, w1, b1, w2, b2, batchp, nmask):
    def body(x_ref, p0_ref, p1_ref, wu_ref, bu_ref, w1_ref, b1_ref,
             w2_ref, b2_ref, b_ref, nm_ref, o_ref):
        i = pl.program_id(0)
        agg = p0_ref[...] + p1_ref[...]
        x = x_ref[...] + jnp.maximum(
            jnp.dot(agg, wu_ref[...], preferred_element_type=_f32)
            + bu_ref[...], 0.0)
        h = jnp.maximum(
            jnp.dot(x, w1_ref[...], preferred_element_type=_f32)
            + b1_ref[...], 0.0)
        u = (jnp.dot(h, w2_ref[...], preferred_element_type=_f32)
             + b2_ref[...]) * nm_ref[...]
        oh = (b_ref[...] == lax.broadcasted_iota(jnp.int32, (BLK, NG), 1))
        part = lax.dot_general(oh.astype(_f32), u,
                               (((0,), (0,)), ((), ())),
                               preferred_element_type=_f32)

        @pl.when(i == 0)
        def _():
            o_ref[...] = part

        @pl.when(i > 0)
        def _():
            o_ref[...] += part

    nb = pl.BlockSpec((BLK, NSC), lambda i: (i, 0))
    return pl.pallas_call(
        body,
        grid=(NBLK,),
        in_specs=[nb, nb, nb,
                  pl.BlockSpec((NSC, NSC), lambda i: (0, 0)),
                  pl.BlockSpec((1, NSC), lambda i: (0, 0)),
                  pl.BlockSpec((NSC, 3 * NSC), lambda i: (0, 0)),
                  pl.BlockSpec((1, 3 * NSC), lambda i: (0, 0)),
                  pl.BlockSpec((3 * NSC, NCLS), lambda i: (0, 0)),
                  pl.BlockSpec((1, NCLS), lambda i: (0, 0)),
                  pl.BlockSpec((BLK, 1), lambda i: (i, 0)),
                  pl.BlockSpec((BLK, 1), lambda i: (i, 0))],
        out_specs=pl.BlockSpec((NG, NCLS), lambda i: (0, 0)),
        out_shape=jax.ShapeDtypeStruct((NG, NCLS), _f32),
    )(xprev, p0, p1, wu, bu, w1, b1, w2, b2, batchp, nmask)


def _sc_mesh():
    return plsc.VectorSubcoreMesh(core_axis_name="c", subcore_axis_name="s")


_SC_PARAMS = pltpu.CompilerParams(use_tc_tiling_on_sc=False)


def _sc_gather_pos(pc, row2d, col2d):
    @functools.partial(
        pl.kernel,
        out_type=jax.ShapeDtypeStruct((E_PAD, PW), _f32),
        mesh=_sc_mesh(),
        compiler_params=_SC_PARAMS,
        scratch_types=[pltpu.VMEM((IBG, CH), jnp.int32),
                       pltpu.VMEM((IBG, CH), jnp.int32),
                       pltpu.VMEM((CH, PW), _f32),
                       pltpu.VMEM((CH, PW), _f32),
                       pltpu.VMEM((CH, PW), _f32),
                       pltpu.VMEM((CH, PW), _f32),
                       pltpu.SemaphoreType.DMA,
                       pltpu.SemaphoreType.DMA,
                       pltpu.SemaphoreType.DMA,
                       pltpu.SemaphoreType.DMA])
    def k(pc_hbm, row_hbm, col_hbm, dv_hbm,
          ridx, cidx, rb0, rb1, cb0, cb1, s10, s11, s20, s21):
        wid = lax.axis_index("c") * NTILES + lax.axis_index("s")
        rbs, cbs = (rb0, rb1), (cb0, cb1)
        s1s, s2s = (s10, s11), (s20, s21)

        @pl.loop(0, NSLABG)
        def _(t):
            pltpu.sync_copy(row_hbm.at[pl.ds(wid * CPT + t * IBG, IBG)], ridx)
            pltpu.sync_copy(col_hbm.at[pl.ds(wid * CPT + t * IBG, IBG)], cidx)

            def start(b):
                sl = b % 2
                da = pltpu.async_copy(pc_hbm.at[ridx.at[b]], rbs[sl], s1s[sl])
                db = pltpu.async_copy(pc_hbm.at[cidx.at[b]], cbs[sl], s2s[sl])
                return (da, db)

            pend = start(0)
            for b in range(IBG):
                sl = b % 2
                nxt = start(b + 1) if b + 1 < IBG else None
                for d in pend:
                    d.wait()
                rb, cb = rbs[sl], cbs[sl]

                @plsc.parallel_loop(0, CH, unroll=8)
                def _(r):
                    s_ = pl.ds(0, 16)
                    rb[r, s_] = rb[r, s_] - cb[r, s_]

                base = (wid * CPT + t * IBG + b) * CH
                pltpu.sync_copy(rb, dv_hbm.at[pl.ds(base, CH)])
                pend = nxt

    return k(pc, row2d, col2d)


def _zero_acc(zb, acc, sid):
    zeros16 = jnp.zeros((16,), _f32)

    @pl.loop(0, ZR)
    def _(r):
        zb[r, pl.ds(0, 16)] = zeros16
        zb[r, pl.ds(16, 16)] = zeros16

    @pl.loop(0, NPT // ZR)
    def _(t):
        pltpu.sync_copy(zb, acc.at[pl.ds(sid * NPT + t * ZR, ZR)])


def _sc_scatter_he(he, row2d):
    @functools.partial(
        pl.kernel,
        out_type=jax.ShapeDtypeStruct((NCORES, N_PAD, NSC), _f32),
        mesh=_sc_mesh(),
        compiler_params=_SC_PARAMS,
        scratch_types=[pltpu.VMEM((IBG, CH), jnp.int32),
                       pltpu.VMEM((CH, NSC), _f32),
                       pltpu.VMEM((CH, NSC), _f32),
                       pltpu.VMEM((ZR, NSC), _f32),
                       pltpu.VMEM_SHARED((N_PAD, NSC), _f32),
                       pltpu.SemaphoreType.DMA,
                       pltpu.SemaphoreType.DMA])
    def k(he_hbm, row_hbm, out_hbm, ridx, hb0, hb1, zb, acc, s0, s1):
        cid = lax.axis_index("c")
        sid = lax.axis_index("s")
        wid = cid * NTILES + sid
        hbs, ss = (hb0, hb1), (s0, s1)
        _zero_acc(zb, acc, sid)
        plsc.subcore_barrier()

        @pl.loop(0, NSLABG)
        def _(t):
            pltpu.sync_copy(row_hbm.at[pl.ds(wid * CPT + t * IBG, IBG)], ridx)

            def start(b):
                sl = b % 2
                base = (wid * CPT + t * IBG + b) * CH
                return pltpu.async_copy(he_hbm.at[pl.ds(base, CH)],
                                        hbs[sl], ss[sl])

            pend = start(0)
            for b in range(IBG):
                sl = b % 2
                nxt = start(b + 1) if b + 1 < IBG else None
                pend.wait()
                pltpu.sync_copy(hbs[sl], acc.at[ridx.at[b]], add=True)
                pend = nxt

        plsc.subcore_barrier()
        pltpu.sync_copy(acc.at[pl.ds(sid * NPT, NPT)],
                        out_hbm.at[cid].at[pl.ds(sid * NPT, NPT)])

    return k(he, row2d)


def _sc_layer(a, b, c, row2d, col2d):
    @functools.partial(
        pl.kernel,
        out_type=jax.ShapeDtypeStruct((NCORES, N_PAD, NSC), _f32),
        mesh=_sc_mesh(),
        compiler_params=_SC_PARAMS,
        scratch_types=[pltpu.VMEM((IBL, CH), jnp.int32),
                       pltpu.VMEM((IBL, CH), jnp.int32),
                       pltpu.VMEM((CH, NSC), _f32),
                       pltpu.VMEM((CH, NSC), _f32),
                       pltpu.VMEM((CH, NSC), _f32),
                       pltpu.VMEM((CH, NSC), _f32),
                       pltpu.VMEM((CH, NSC), _f32),
                       pltpu.VMEM((CH, NSC), _f32),
                       pltpu.VMEM((ZR, NSC), _f32),
                       pltpu.VMEM_SHARED((N_PAD, NSC), _f32),
                       pltpu.SemaphoreType.DMA,
                       pltpu.SemaphoreType.DMA,
                       pltpu.SemaphoreType.DMA,
                       pltpu.SemaphoreType.DMA,
                       pltpu.SemaphoreType.DMA,
                       pltpu.SemaphoreType.DMA])
    def k(a_hbm, b_hbm, c_hbm, row_hbm, col_hbm, out_hbm,
          ridx, cidx, ab0, ab1, bb0, bb1, cb0, cb1, zb, acc,
          sa0, sa1, sb0, sb1, sc0, sc1):
        cid = lax.axis_index("c")
        sid = lax.axis_index("s")
        wid = cid * NTILES + sid
        abs_, bbs, cbs = (ab0, ab1), (bb0, bb1), (cb0, cb1)
        sas, sbs, scs = (sa0, sa1), (sb0, sb1), (sc0, sc1)
        _zero_acc(zb, acc, sid)
        plsc.subcore_barrier()

        @pl.loop(0, NSLABL)
        def _(t):
            pltpu.sync_copy(row_hbm.at[pl.ds(wid * CPT + t * IBL, IBL)], ridx)
            pltpu.sync_copy(col_hbm.at[pl.ds(wid * CPT + t * IBL, IBL)], cidx)

            def start(b):
                sl = b % 2
                base = (wid * CPT + t * IBL + b) * CH
                da = pltpu.async_copy(a_hbm.at[ridx.at[b]], abs_[sl], sas[sl])
                db = pltpu.async_copy(b_hbm.at[cidx.at[b]], bbs[sl], sbs[sl])
                dc = pltpu.async_copy(c_hbm.at[pl.ds(base, CH)],
                                      cbs[sl], scs[sl])
                return (da, db, dc)

            pend = start(0)
            for b in range(IBL):
                sl = b % 2
                nxt = start(b + 1) if b + 1 < IBL else None
                for d in pend:
                    d.wait()
                ab, bb, cb = abs_[sl], bbs[sl], cbs[sl]

                @plsc.parallel_loop(0, CH, unroll=8)
                def _(r):
                    for c0 in (0, 16):
                        s_ = pl.ds(c0, 16)
                        ab[r, s_] = jnp.maximum(
                            ab[r, s_] + bb[r, s_] + cb[r, s_], 0.0)

                pltpu.sync_copy(ab, acc.at[ridx.at[b]], add=True)
                pend = nxt

        plsc.subcore_barrier()
        pltpu.sync_copy(acc.at[pl.ds(sid * NPT, NPT)],
                        out_hbm.at[cid].at[pl.ds(sid * NPT, NPT)])

    return k(a, b, c, row2d, col2d)


def kernel(pos, edge_index, batch, W_emb, b_emb, W_msg, b_msg, W_upd, b_upd,
           W1, b1, W2, b2):
    row = edge_index[0].astype(jnp.int32)
    col = edge_index[1].astype(jnp.int32)
    batch = batch.astype(jnp.int32)

    ech = E // CH
    padc = jnp.full((E_PAD // CH - ech, CH), N, jnp.int32)
    row2d = jnp.concatenate([row.reshape(ech, CH), padc])
    col2d = jnp.concatenate([col.reshape(ech, CH), padc])

    posw = jnp.zeros((N_PAD, PW), _f32)
    posw = posw.at[:N, :3].set(pos).at[:N, 3].set(1.0)
    batchp = jnp.zeros((N_PAD, 1), jnp.int32).at[:N, 0].set(batch)
    nmask = (jnp.arange(N_PAD) < N).astype(_f32)[:, None]
    emask = (jnp.arange(E_PAD) < E).astype(_f32)[:, None]

    w_emb = W_emb.reshape(1, NSC)
    b_emb2 = b_emb.reshape(1, NSC)
    wd = [W_msg[i, :NSC, :] for i in range(NL)]
    ws = [W_msg[i, NSC:2 * NSC, :] for i in range(NL)]
    wr = [W_msg[i, 2 * NSC:, :] for i in range(NL)]
    bm = [b_msg[i].reshape(1, NSC) for i in range(NL)]
    wu = [W_upd[i] for i in range(NL)]
    bu = [b_upd[i].reshape(1, NSC) for i in range(NL)]
    b1_2 = b1.reshape(1, 3 * NSC)
    b2_2 = b2.reshape(1, NCLS)

    sums = _graph_sums(posw, batchp)
    pc = _center(sums, posw, batchp)

    dv = _sc_gather_pos(pc, row2d, col2d)
    he, c1, c2, c3 = _edge_feats(dv, emask, w_emb, b_emb2, wr, bm)
    cs = [c1, c2, c3]

    parts = _sc_scatter_he(he, row2d)
    x, a_n, b_n = _combine0_proj(parts[0], parts[1], wd[0], ws[0])

    for i in range(NL):
        parts = _sc_layer(a_n, b_n, cs[i], row2d, col2d)
        if i < NL - 1:
            x, a_n, b_n = _update_proj(x, parts[0], parts[1], wu[i], bu[i],
                                       wd[i + 1], ws[i + 1])
        else:
            out = _readout(x, parts[0], parts[1], wu[i], bu[i],
                           W1, b1_2, W2, b2_2, batchp, nmask)
    return out
